# baseline (speedup 1.0000x reference)
import functools

import jax
import jax.numpy as jnp
from jax import lax
from jax.experimental import pallas as pl
from jax.experimental.pallas import tpu as pltpu
from jax.experimental.pallas import tpu_sc as plsc

_N = 100000
_NUM_NODES = 10000
_D = 688
_K = 6
_NW = 32
_NODES_PAD = 10240
_NPW = _NODES_PAD // _NW
_NB = 8
_NBATCH = _NPW // _NB
_ROWS = _NB * _K
_C16 = _D // 16

_mesh = plsc.VectorSubcoreMesh(core_axis_name="c", subcore_axis_name="s")


@functools.partial(
    pl.kernel,
    out_type=(
        jax.ShapeDtypeStruct((_NODES_PAD, _D), jnp.float32),
        jax.ShapeDtypeStruct((_NODES_PAD,), jnp.float32),
    ),
    mesh=_mesh,
    compiler_params=pltpu.CompilerParams(use_tc_tiling_on_sc=False),
    scratch_types=[
        pltpu.VMEM((_NPW * _K,), jnp.int32),
        pltpu.VMEM((_NPW * _K,), jnp.float32),
        pltpu.VMEM((_NPW,), jnp.int32),
        pltpu.VMEM((_NPW,), jnp.float32),
        pltpu.VMEM((32, 16), jnp.float32),
        pltpu.VMEM((_ROWS, _D), jnp.float32),
        pltpu.VMEM((_ROWS, _D), jnp.float32),
        pltpu.VMEM((_NB, _D), jnp.float32),
        pltpu.VMEM((_NB, _D), jnp.float32),
        pltpu.SemaphoreType.DMA,
        pltpu.SemaphoreType.DMA,
        pltpu.SemaphoreType.DMA,
    ],
)
def _sc_aggregate(msg_hbm, ts_hbm, ridx_hbm, tsi_hbm, mask_hbm, w_hbm,
                  out_msg, out_ts,
                  ridx_v, mask_v, tsi_v, tsv_v, w_v,
                  rows0, rows1, ob0, ob1, sem0, sem1, sem_ts):
    wid = lax.axis_index("s") * 2 + lax.axis_index("c")
    nbase = wid * _NPW

    pltpu.sync_copy(ridx_hbm.at[pl.ds(nbase * _K, _NPW * _K)], ridx_v)
    pltpu.sync_copy(mask_hbm.at[pl.ds(nbase * _K, _NPW * _K)], mask_v)
    pltpu.sync_copy(tsi_hbm.at[pl.ds(nbase, _NPW)], tsi_v)
    pltpu.sync_copy(w_hbm, w_v)

    pltpu.async_copy(ts_hbm.at[tsi_v], tsv_v, sem_ts).wait()
    pltpu.sync_copy(tsv_v, out_ts.at[pl.ds(nbase, _NPW)])

    rows_bufs = (rows0, rows1)
    obufs = (ob0, ob1)
    sems = (sem0, sem1)

    def _fire(b, t):
        pltpu.make_async_copy(
            msg_hbm.at[ridx_v.at[pl.ds(b * _ROWS, _ROWS)]],
            rows_bufs[t], sems[t]).start()

    def _wait(t):
        pltpu.make_async_copy(
            msg_hbm.at[ridx_v.at[pl.ds(0, _ROWS)]],
            rows_bufs[t], sems[t]).wait()

    w = [w_v[i] for i in range(32)]

    def _compute(b, t):
        rows = rows_bufs[t]
        ob = obufs[t]
        mv = [mask_v[pl.ds(b * _ROWS + 16 * i, 16)] for i in range(3)]
        for j in range(_NB):
            m = [jnp.full((16,), mv[(j * _K + s) // 16][(j * _K + s) % 16],
                          jnp.float32)
                 for s in range(_K)]

            def body(c, carry):
                sl = pl.ds(c * 16, 16)
                x = [rows[j * _K + s, sl] * m[s] for s in range(_K)]
                h = []
                for jj in range(3):
                    acc = w[18 + jj]
                    for s in range(_K):
                        acc = acc + x[s] * w[s * 3 + jj]
                    h.append(jnp.maximum(acc, 0.0))
                g = []
                for kk in range(2):
                    acc = w[27 + kk]
                    for jj in range(3):
                        acc = acc + h[jj] * w[21 + jj * 2 + kk]
                    g.append(jnp.maximum(acc, 0.0))
                o = w[31] + g[0] * w[29] + g[1] * w[30]
                ob[j, sl] = o
                return carry

            lax.fori_loop(0, _C16, body, 0)
        pltpu.sync_copy(ob, out_msg.at[pl.ds(nbase + b * _NB, _NB)])

    _fire(0, 0)

    def outer(i, carry):
        b0 = i * 2
        _wait(0)
        _fire(b0 + 1, 1)
        _compute(b0, 0)
        b1 = b0 + 1
        _wait(1)

        @pl.when(b1 + 1 < _NBATCH)
        def _():
            _fire(b1 + 1, 0)

        _compute(b1, 1)
        return carry

    lax.fori_loop(0, _NBATCH // 2, outer, 0)


def kernel(messages, timestamps, segment_ids, W1, b1, W2, b2, W3, b3):
    nodes = jnp.arange(_NUM_NODES, dtype=segment_ids.dtype)
    right = jnp.searchsorted(segment_ids, nodes, side="right").astype(jnp.int32)
    left = jnp.searchsorted(segment_ids, nodes, side="left").astype(jnp.int32)
    cnt = jnp.minimum(right - left, _K)
    last_idx = right - 1
    ts_idx = jnp.clip(last_idx, 0, _N - 1)
    s = jnp.arange(_K, dtype=jnp.int32)
    rows = jnp.clip(last_idx[:, None] - (_K - 1 - s)[None, :], 0, _N - 1)
    validf = ((_K - 1 - s)[None, :] < cnt[:, None]).astype(jnp.float32)

    pad = _NODES_PAD - _NUM_NODES
    row_idx = jnp.concatenate([rows.reshape(-1),
                               jnp.zeros((pad * _K,), jnp.int32)])
    maskf = jnp.concatenate([validf.reshape(-1),
                             jnp.zeros((pad * _K,), jnp.float32)])
    tsi = jnp.concatenate([ts_idx, jnp.zeros((pad,), jnp.int32)])

    wflat = jnp.concatenate([W1.reshape(-1), b1, W2.reshape(-1), b2,
                             W3.reshape(-1), b3])
    wrep = jnp.tile(wflat[:, None], (1, 16))

    msg_tbl = jnp.minimum(messages, jnp.float32(3.0e38))
    out_msg, out_ts = _sc_aggregate(msg_tbl, timestamps, row_idx, tsi,
                                    maskf, wrep)
    return out_msg[:_NUM_NODES], out_ts[:_NUM_NODES]


# speedup vs baseline: 1.6745x; 1.6745x over previous
import functools

import jax
import jax.numpy as jnp
from jax import lax
from jax.experimental import pallas as pl
from jax.experimental.pallas import tpu as pltpu
from jax.experimental.pallas import tpu_sc as plsc

_N = 100000
_NUM_NODES = 10000
_D = 688
_K = 6
_NW = 32
_NODES_PAD = 10240
_NPW = _NODES_PAD // _NW
_NB = 8
_NBATCH = _NPW // _NB
_ROWS = _NB * _K
_C16 = _D // 16

_mesh = plsc.VectorSubcoreMesh(core_axis_name="c", subcore_axis_name="s")


@functools.partial(
    pl.kernel,
    out_type=(
        jax.ShapeDtypeStruct((_NODES_PAD, _D), jnp.float32),
        jax.ShapeDtypeStruct((_NODES_PAD,), jnp.float32),
    ),
    mesh=_mesh,
    compiler_params=pltpu.CompilerParams(use_tc_tiling_on_sc=False),
    scratch_types=[
        pltpu.VMEM((_NPW * _K,), jnp.int32),
        pltpu.VMEM((_NPW * _K,), jnp.float32),
        pltpu.VMEM((_NPW,), jnp.int32),
        pltpu.VMEM((_NPW,), jnp.float32),
        pltpu.VMEM((32, 16), jnp.float32),
        pltpu.VMEM((_ROWS, _D), jnp.float32),
        pltpu.VMEM((_ROWS, _D), jnp.float32),
        pltpu.VMEM((_NB, _D), jnp.float32),
        pltpu.VMEM((_NB, _D), jnp.float32),
        pltpu.SemaphoreType.DMA,
        pltpu.SemaphoreType.DMA,
        pltpu.SemaphoreType.DMA,
    ],
)
def _sc_aggregate(msg_hbm, ts_hbm, ridx_hbm, tsi_hbm, mask_hbm, w_hbm,
                  out_msg, out_ts,
                  ridx_v, mask_v, tsi_v, tsv_v, w_v,
                  rows0, rows1, ob0, ob1, sem0, sem1, sem_ts):
    wid = lax.axis_index("s") * 2 + lax.axis_index("c")
    nbase = wid * _NPW

    pltpu.sync_copy(ridx_hbm.at[pl.ds(nbase * _K, _NPW * _K)], ridx_v)
    pltpu.sync_copy(mask_hbm.at[pl.ds(nbase * _K, _NPW * _K)], mask_v)
    pltpu.sync_copy(tsi_hbm.at[pl.ds(nbase, _NPW)], tsi_v)
    pltpu.sync_copy(w_hbm, w_v)

    pltpu.async_copy(ts_hbm.at[tsi_v], tsv_v, sem_ts).wait()
    pltpu.sync_copy(tsv_v, out_ts.at[pl.ds(nbase, _NPW)])

    rows_bufs = (rows0, rows1)
    obufs = (ob0, ob1)
    sems = (sem0, sem1)

    def _fire(b, t):
        pltpu.make_async_copy(
            msg_hbm.at[ridx_v.at[pl.ds(b * _ROWS, _ROWS)]],
            rows_bufs[t], sems[t]).start()

    def _wait(t):
        pltpu.make_async_copy(
            msg_hbm.at[ridx_v.at[pl.ds(0, _ROWS)]],
            rows_bufs[t], sems[t]).wait()

    w = [w_v[i] for i in range(32)]

    def _compute(b, t):
        rows = rows_bufs[t]
        ob = obufs[t]
        mv = [mask_v[pl.ds(b * _ROWS + 16 * i, 16)] for i in range(3)]
        for j in range(_NB):
            m = [jnp.full((16,), mv[(j * _K + s) // 16][(j * _K + s) % 16],
                          jnp.float32)
                 for s in range(_K)]

            def body(c, carry):
                sl = pl.ds(c * 16, 16)
                x = [rows[j * _K + s, sl] * m[s] for s in range(_K)]
                h = []
                for jj in range(3):
                    acc = w[18 + jj]
                    for s in range(_K):
                        acc = acc + x[s] * w[s * 3 + jj]
                    h.append(jnp.maximum(acc, 0.0))
                g = []
                for kk in range(2):
                    acc = w[27 + kk]
                    for jj in range(3):
                        acc = acc + h[jj] * w[21 + jj * 2 + kk]
                    g.append(jnp.maximum(acc, 0.0))
                o = w[31] + g[0] * w[29] + g[1] * w[30]
                ob[j, sl] = o
                return carry

            lax.fori_loop(0, _C16, body, 0)
        pltpu.sync_copy(ob, out_msg.at[pl.ds(nbase + b * _NB, _NB)])

    _fire(0, 0)

    def outer(i, carry):
        b0 = i * 2
        _wait(0)
        _fire(b0 + 1, 1)
        _compute(b0, 0)
        b1 = b0 + 1
        _wait(1)

        @pl.when(b1 + 1 < _NBATCH)
        def _():
            _fire(b1 + 1, 0)

        _compute(b1, 1)
        return carry

    lax.fori_loop(0, _NBATCH // 2, outer, 0)


_TB = 512


def _tc_transpose_body(x_ref, o_ref):
    o_ref[...] = jnp.transpose(x_ref[...], (1, 0))


def _tc_row_majorize(m_t):
    grid = (pl.cdiv(_N, _TB),)
    return pl.pallas_call(
        _tc_transpose_body,
        grid=grid,
        in_specs=[pl.BlockSpec((_D, _TB), lambda i: (0, i))],
        out_specs=pl.BlockSpec((_TB, _D), lambda i: (i, 0)),
        out_shape=jax.ShapeDtypeStruct((_N, _D), jnp.float32),
    )(m_t)


def kernel(messages, timestamps, segment_ids, W1, b1, W2, b2, W3, b3):
    nodes = jnp.arange(_NUM_NODES, dtype=segment_ids.dtype)
    right = jnp.searchsorted(segment_ids, nodes, side="right").astype(jnp.int32)
    left = jnp.searchsorted(segment_ids, nodes, side="left").astype(jnp.int32)
    cnt = jnp.minimum(right - left, _K)
    last_idx = right - 1
    ts_idx = jnp.clip(last_idx, 0, _N - 1)
    s = jnp.arange(_K, dtype=jnp.int32)
    rows = jnp.clip(last_idx[:, None] - (_K - 1 - s)[None, :], 0, _N - 1)
    validf = ((_K - 1 - s)[None, :] < cnt[:, None]).astype(jnp.float32)

    pad = _NODES_PAD - _NUM_NODES
    row_idx = jnp.concatenate([rows.reshape(-1),
                               jnp.zeros((pad * _K,), jnp.int32)])
    maskf = jnp.concatenate([validf.reshape(-1),
                             jnp.zeros((pad * _K,), jnp.float32)])
    tsi = jnp.concatenate([ts_idx, jnp.zeros((pad,), jnp.int32)])

    wflat = jnp.concatenate([W1.reshape(-1), b1, W2.reshape(-1), b2,
                             W3.reshape(-1), b3])
    wrep = jnp.tile(wflat[:, None], (1, 16))

    msg_tbl = _tc_row_majorize(messages.T)
    out_msg, out_ts = _sc_aggregate(msg_tbl, timestamps, row_idx, tsi,
                                    maskf, wrep)
    return out_msg[:_NUM_NODES], out_ts[:_NUM_NODES]


# speedup vs baseline: 1.7375x; 1.0377x over previous
import functools

import jax
import jax.numpy as jnp
from jax import lax
from jax.experimental import pallas as pl
from jax.experimental.pallas import tpu as pltpu
from jax.experimental.pallas import tpu_sc as plsc

_N = 100000
_NUM_NODES = 10000
_D = 688
_K = 6
_NW = 32
_NODES_PAD = 10240
_NPW = _NODES_PAD // _NW
_NB = 8
_NBATCH = _NPW // _NB
_ROWS = _NB * _K
_C16 = _D // 16

_mesh = plsc.VectorSubcoreMesh(core_axis_name="c", subcore_axis_name="s")


@functools.partial(
    pl.kernel,
    out_type=(
        jax.ShapeDtypeStruct((_NODES_PAD, _D), jnp.float32),
        jax.ShapeDtypeStruct((_NODES_PAD,), jnp.float32),
    ),
    mesh=_mesh,
    compiler_params=pltpu.CompilerParams(use_tc_tiling_on_sc=False),
    scratch_types=[
        pltpu.VMEM((_NPW * _K,), jnp.int32),
        pltpu.VMEM((_NPW * _K,), jnp.float32),
        pltpu.VMEM((_NPW,), jnp.int32),
        pltpu.VMEM((_NPW,), jnp.float32),
        pltpu.VMEM((32, 16), jnp.float32),
        pltpu.VMEM((_ROWS, _D), jnp.float32),
        pltpu.VMEM((_ROWS, _D), jnp.float32),
        pltpu.VMEM((_NB, _D), jnp.float32),
        pltpu.VMEM((_NB, _D), jnp.float32),
        pltpu.SemaphoreType.DMA,
        pltpu.SemaphoreType.DMA,
        pltpu.SemaphoreType.DMA,
    ],
)
def _sc_aggregate(msg_hbm, ts_hbm, ridx_hbm, tsi_hbm, mask_hbm, w_hbm,
                  out_msg, out_ts,
                  ridx_v, mask_v, tsi_v, tsv_v, w_v,
                  rows0, rows1, ob0, ob1, sem0, sem1, sem_ts):
    wid = lax.axis_index("s") * 2 + lax.axis_index("c")
    nbase = wid * _NPW

    pltpu.sync_copy(ridx_hbm.at[pl.ds(nbase * _K, _NPW * _K)], ridx_v)
    pltpu.sync_copy(mask_hbm.at[pl.ds(nbase * _K, _NPW * _K)], mask_v)
    pltpu.sync_copy(tsi_hbm.at[pl.ds(nbase, _NPW)], tsi_v)
    pltpu.sync_copy(w_hbm, w_v)

    pltpu.async_copy(ts_hbm.at[tsi_v], tsv_v, sem_ts).wait()
    pltpu.sync_copy(tsv_v, out_ts.at[pl.ds(nbase, _NPW)])

    rows_bufs = (rows0, rows1)
    obufs = (ob0, ob1)
    sems = (sem0, sem1)

    def _fire(b, t):
        pltpu.make_async_copy(
            msg_hbm.at[ridx_v.at[pl.ds(b * _ROWS, _ROWS)]],
            rows_bufs[t], sems[t]).start()

    def _wait(t):
        pltpu.make_async_copy(
            msg_hbm.at[ridx_v.at[pl.ds(0, _ROWS)]],
            rows_bufs[t], sems[t]).wait()

    w = [w_v[i] for i in range(32)]

    def _compute(b, t):
        rows = rows_bufs[t]
        ob = obufs[t]
        mv = [mask_v[pl.ds(b * _ROWS + 16 * i, 16)] for i in range(3)]
        for j in range(_NB):
            m = [jnp.full((16,), mv[(j * _K + s) // 16][(j * _K + s) % 16],
                          jnp.float32)
                 for s in range(_K)]

            def body(c, carry):
                sl = pl.ds(c * 16, 16)
                x = [rows[j * _K + s, sl] * m[s] for s in range(_K)]
                h = []
                for jj in range(3):
                    acc = w[18 + jj]
                    for s in range(_K):
                        acc = acc + x[s] * w[s * 3 + jj]
                    h.append(jnp.maximum(acc, 0.0))
                g = []
                for kk in range(2):
                    acc = w[27 + kk]
                    for jj in range(3):
                        acc = acc + h[jj] * w[21 + jj * 2 + kk]
                    g.append(jnp.maximum(acc, 0.0))
                o = w[31] + g[0] * w[29] + g[1] * w[30]
                ob[j, sl] = o
                return carry

            lax.fori_loop(0, _C16, body, 0)
        pltpu.sync_copy(ob, out_msg.at[pl.ds(nbase + b * _NB, _NB)])

    _fire(0, 0)

    def outer(i, carry):
        b0 = i * 2
        _wait(0)
        _fire(b0 + 1, 1)
        _compute(b0, 0)
        b1 = b0 + 1
        _wait(1)

        @pl.when(b1 + 1 < _NBATCH)
        def _():
            _fire(b1 + 1, 0)

        _compute(b1, 1)
        return carry

    lax.fori_loop(0, _NBATCH // 2, outer, 0)


_TB = 1024


def _tc_transpose_body(x_ref, o_ref):
    o_ref[...] = jnp.transpose(x_ref[...], (1, 0))


def _tc_row_majorize(m_t):
    grid = (pl.cdiv(_N, _TB),)
    return pl.pallas_call(
        _tc_transpose_body,
        grid=grid,
        in_specs=[pl.BlockSpec((_D, _TB), lambda i: (0, i))],
        out_specs=pl.BlockSpec((_TB, _D), lambda i: (i, 0)),
        out_shape=jax.ShapeDtypeStruct((_N, _D), jnp.float32),
    )(m_t)


def kernel(messages, timestamps, segment_ids, W1, b1, W2, b2, W3, b3):
    nodes = jnp.arange(_NUM_NODES, dtype=segment_ids.dtype)
    right = jnp.searchsorted(segment_ids, nodes, side="right").astype(jnp.int32)
    left = jnp.searchsorted(segment_ids, nodes, side="left").astype(jnp.int32)
    cnt = jnp.minimum(right - left, _K)
    last_idx = right - 1
    ts_idx = jnp.clip(last_idx, 0, _N - 1)
    s = jnp.arange(_K, dtype=jnp.int32)
    rows = jnp.clip(last_idx[:, None] - (_K - 1 - s)[None, :], 0, _N - 1)
    validf = ((_K - 1 - s)[None, :] < cnt[:, None]).astype(jnp.float32)

    pad = _NODES_PAD - _NUM_NODES
    row_idx = jnp.concatenate([rows.reshape(-1),
                               jnp.zeros((pad * _K,), jnp.int32)])
    maskf = jnp.concatenate([validf.reshape(-1),
                             jnp.zeros((pad * _K,), jnp.float32)])
    tsi = jnp.concatenate([ts_idx, jnp.zeros((pad,), jnp.int32)])

    wflat = jnp.concatenate([W1.reshape(-1), b1, W2.reshape(-1), b2,
                             W3.reshape(-1), b3])
    wrep = jnp.tile(wflat[:, None], (1, 16))

    msg_tbl = _tc_row_majorize(messages.T)
    out_msg, out_ts = _sc_aggregate(msg_tbl, timestamps, row_idx, tsi,
                                    maskf, wrep)
    return out_msg[:_NUM_NODES], out_ts[:_NUM_NODES]


# speedup vs baseline: 2.6729x; 1.5384x over previous
import functools

import jax
import jax.numpy as jnp
from jax import lax
from jax.experimental import pallas as pl
from jax.experimental.pallas import tpu as pltpu
from jax.experimental.pallas import tpu_sc as plsc

_N = 100000
_NUM_NODES = 10000
_D = 688
_K = 6
_NW = 32
_NODES_PAD = 10240
_NPW = _NODES_PAD // _NW
_NB = 8
_NBATCH = _NPW // _NB
_ROWS = _NB * _K
_C16 = _D // 16

_mesh = plsc.VectorSubcoreMesh(core_axis_name="c", subcore_axis_name="s")


@functools.partial(
    pl.kernel,
    out_type=(
        jax.ShapeDtypeStruct((_NODES_PAD, _D), jnp.float32),
        jax.ShapeDtypeStruct((_NODES_PAD,), jnp.float32),
    ),
    mesh=_mesh,
    compiler_params=pltpu.CompilerParams(use_tc_tiling_on_sc=False,
                                        needs_layout_passes=False),
    scratch_types=[
        pltpu.VMEM((_NPW * _K,), jnp.int32),
        pltpu.VMEM((_NPW * _K,), jnp.float32),
        pltpu.VMEM((_NPW,), jnp.int32),
        pltpu.VMEM((_NPW,), jnp.float32),
        pltpu.VMEM((32, 16), jnp.float32),
        pltpu.VMEM((_ROWS, _D), jnp.float32),
        pltpu.VMEM((_ROWS, _D), jnp.float32),
        pltpu.VMEM((_NB, _D), jnp.float32),
        pltpu.VMEM((_NB, _D), jnp.float32),
        pltpu.SemaphoreType.DMA,
        pltpu.SemaphoreType.DMA,
        pltpu.SemaphoreType.DMA,
    ],
)
def _sc_aggregate(msg_hbm, ts_hbm, ridx_hbm, tsi_hbm, mask_hbm, w_hbm,
                  out_msg, out_ts,
                  ridx_v, mask_v, tsi_v, tsv_v, w_v,
                  rows0, rows1, ob0, ob1, sem0, sem1, sem_ts):
    wid = lax.axis_index("s") * 2 + lax.axis_index("c")
    nbase = wid * _NPW

    pltpu.sync_copy(ridx_hbm.at[pl.ds(nbase * _K, _NPW * _K)], ridx_v)
    pltpu.sync_copy(mask_hbm.at[pl.ds(nbase * _K, _NPW * _K)], mask_v)
    pltpu.sync_copy(tsi_hbm.at[pl.ds(nbase, _NPW)], tsi_v)
    pltpu.sync_copy(w_hbm, w_v)

    pltpu.async_copy(ts_hbm.at[tsi_v], tsv_v, sem_ts).wait()
    pltpu.sync_copy(tsv_v, out_ts.at[pl.ds(nbase, _NPW)])

    rows_bufs = (rows0, rows1)
    obufs = (ob0, ob1)
    sems = (sem0, sem1)

    def _fire(b, t):
        pltpu.make_async_copy(
            msg_hbm.at[ridx_v.at[pl.ds(b * _ROWS, _ROWS)]],
            rows_bufs[t], sems[t]).start()

    def _wait(t):
        pltpu.make_async_copy(
            msg_hbm.at[ridx_v.at[pl.ds(0, _ROWS)]],
            rows_bufs[t], sems[t]).wait()

    w = [w_v[i] for i in range(32)]

    def _compute(b, t):
        rows = rows_bufs[t]
        ob = obufs[t]
        mv = [mask_v[pl.ds(b * _ROWS + 16 * i, 16)] for i in range(3)]
        for j in range(_NB):
            m = [jnp.full((16,), mv[(j * _K + s) // 16][(j * _K + s) % 16],
                          jnp.float32)
                 for s in range(_K)]

            def body(c, carry):
                sl = pl.ds(c * 16, 16)
                x = [rows[j * _K + s, sl] * m[s] for s in range(_K)]
                h = []
                for jj in range(3):
                    acc = w[18 + jj]
                    for s in range(_K):
                        acc = acc + x[s] * w[s * 3 + jj]
                    h.append(jnp.maximum(acc, 0.0))
                g = []
                for kk in range(2):
                    acc = w[27 + kk]
                    for jj in range(3):
                        acc = acc + h[jj] * w[21 + jj * 2 + kk]
                    g.append(jnp.maximum(acc, 0.0))
                o = w[31] + g[0] * w[29] + g[1] * w[30]
                ob[j, sl] = o
                return carry

            lax.fori_loop(0, _C16, body, 0)
        pltpu.sync_copy(ob, out_msg.at[pl.ds(nbase + b * _NB, _NB)])

    _fire(0, 0)

    def outer(i, carry):
        b0 = i * 2
        _wait(0)
        _fire(b0 + 1, 1)
        _compute(b0, 0)
        b1 = b0 + 1
        _wait(1)

        @pl.when(b1 + 1 < _NBATCH)
        def _():
            _fire(b1 + 1, 0)

        _compute(b1, 1)
        return carry

    lax.fori_loop(0, _NBATCH // 2, outer, 0)


_BSTRIDE = 3120
_BWIN = 3136
_TAILLO = _N - _BWIN


def _lane_gather(v, idx):
    dn = lax.GatherDimensionNumbers(offset_dims=(), collapsed_slice_dims=(0,),
                                    start_index_map=(0,))
    return lax.gather(v, idx[:, None], dn, (1,),
                      mode=lax.GatherScatterMode.PROMISE_IN_BOUNDS)


@functools.partial(
    pl.kernel,
    out_type=jax.ShapeDtypeStruct((_NW, _NODES_PAD), jnp.int32),
    mesh=_mesh,
    compiler_params=pltpu.CompilerParams(use_tc_tiling_on_sc=False,
                                        needs_layout_passes=False),
    scratch_types=[
        pltpu.VMEM((_BWIN,), jnp.int32),
        pltpu.VMEM((_NODES_PAD,), jnp.int32),
    ],
)
def _sc_boundaries(seg_hbm, out_hbm, seg_v, loc_v):
    wid = lax.axis_index("s") * 2 + lax.axis_index("c")
    iota = lax.iota(jnp.int32, 16)

    def init(i, c):
        loc_v[pl.ds(i * 16, 16)] = jnp.full((16,), -1, jnp.int32)
        return c

    lax.fori_loop(0, _NODES_PAD // 16, init, 0)

    def scan_window(lo):
        pltpu.sync_copy(seg_hbm.at[pl.ds(lo, _BWIN)], seg_v)

        def body(k, prev):
            cur = seg_v[pl.ds(k * 16, 16)]
            rotl = _lane_gather(prev, (iota + 1) & 15)
            nxt = jnp.where(iota < 15, rotl, _lane_gather(cur, iota * 0))
            m = prev != nxt
            ivec = iota + (lo + (k - 1) * 16)
            plsc.store_scatter(loc_v, [prev], ivec, mask=m)
            return cur

        return lax.fori_loop(1, _BWIN // 16, body, seg_v[pl.ds(0, 16)])

    scan_window(wid * _BSTRIDE)

    @pl.when(wid == _NW - 1)
    def _():
        last_vreg = scan_window(_TAILLO)
        rotl = _lane_gather(last_vreg, (iota + 1) & 15)
        m = (last_vreg != rotl) & (iota < 15)
        ivec = iota + (_N - 16)
        plsc.store_scatter(loc_v, [last_vreg], ivec, mask=m)
        plsc.store_scatter(loc_v, [_lane_gather(last_vreg, iota * 0 + 15)],
                           iota * 0 + (_N - 1), mask=iota < 1)

    pltpu.sync_copy(loc_v, out_hbm.at[wid])


_TB = 1024


def _tc_transpose_body(x_ref, o_ref):
    o_ref[...] = jnp.transpose(x_ref[...], (1, 0))


def _tc_row_majorize(m_t):
    grid = (pl.cdiv(_N, _TB),)
    return pl.pallas_call(
        _tc_transpose_body,
        grid=grid,
        in_specs=[pl.BlockSpec((_D, _TB), lambda i: (0, i))],
        out_specs=pl.BlockSpec((_TB, _D), lambda i: (i, 0)),
        out_shape=jax.ShapeDtypeStruct((_N, _D), jnp.float32),
    )(m_t)


def kernel(messages, timestamps, segment_ids, W1, b1, W2, b2, W3, b3):
    lastb = jnp.max(_sc_boundaries(segment_ids), axis=0)
    last_sem = lax.cummax(lastb, axis=0)
    ts_idx = jnp.clip(last_sem, 0, _N - 1)
    first = jnp.concatenate([jnp.full((1,), -1, jnp.int32),
                             last_sem[:-1]]) + 1
    cnt = jnp.where(lastb >= 0, jnp.minimum(lastb - first + 1, _K), 0)
    s = jnp.arange(_K, dtype=jnp.int32)
    rows = jnp.clip(lastb[:, None] - (_K - 1 - s)[None, :], 0, _N - 1)
    validf = ((_K - 1 - s)[None, :] < cnt[:, None]).astype(jnp.float32)

    row_idx = rows.reshape(-1)
    maskf = validf.reshape(-1)
    tsi = ts_idx

    wflat = jnp.concatenate([W1.reshape(-1), b1, W2.reshape(-1), b2,
                             W3.reshape(-1), b3])
    wrep = jnp.tile(wflat[:, None], (1, 16))

    msg_tbl = _tc_row_majorize(messages.T)
    out_msg, out_ts = _sc_aggregate(msg_tbl, timestamps, row_idx, tsi,
                                    maskf, wrep)
    return out_msg[:_NUM_NODES], out_ts[:_NUM_NODES]


# speedup vs baseline: 2.9949x; 1.1205x over previous
import functools

import jax
import jax.numpy as jnp
from jax import lax
from jax.experimental import pallas as pl
from jax.experimental.pallas import tpu as pltpu
from jax.experimental.pallas import tpu_sc as plsc

_N = 100000
_NUM_NODES = 10000
_D = 688
_K = 6
_NW = 32
_NODES_PAD = 10240
_NPW = _NODES_PAD // _NW
_NB = 8
_NBATCH = _NPW // _NB
_ROWS = _NB * _K
_C16 = _D // 16

_mesh = plsc.VectorSubcoreMesh(core_axis_name="c", subcore_axis_name="s")


@functools.partial(
    pl.kernel,
    out_type=(
        jax.ShapeDtypeStruct((_NODES_PAD, _D), jnp.float32),
        jax.ShapeDtypeStruct((_NODES_PAD,), jnp.float32),
    ),
    mesh=_mesh,
    compiler_params=pltpu.CompilerParams(use_tc_tiling_on_sc=False,
                                        needs_layout_passes=False),
    scratch_types=[
        pltpu.VMEM((_NPW * _K,), jnp.int32),
        pltpu.VMEM((_NPW * _K,), jnp.float32),
        pltpu.VMEM((_NPW,), jnp.int32),
        pltpu.VMEM((_NPW,), jnp.float32),
        pltpu.VMEM((32, 16), jnp.float32),
        pltpu.VMEM((_ROWS, _D), jnp.float32),
        pltpu.VMEM((_ROWS, _D), jnp.float32),
        pltpu.VMEM((_NB, _D), jnp.float32),
        pltpu.VMEM((_NB, _D), jnp.float32),
        pltpu.SemaphoreType.DMA,
        pltpu.SemaphoreType.DMA,
        pltpu.SemaphoreType.DMA,
    ],
)
def _sc_aggregate(msg_hbm, ts_hbm, ridx_hbm, tsi_hbm, mask_hbm, w_hbm,
                  out_msg, out_ts,
                  ridx_v, mask_v, tsi_v, tsv_v, w_v,
                  rows0, rows1, ob0, ob1, sem0, sem1, sem_ts):
    wid = lax.axis_index("s") * 2 + lax.axis_index("c")
    nbase = wid * _NPW

    pltpu.sync_copy(ridx_hbm.at[pl.ds(nbase * _K, _NPW * _K)], ridx_v)
    pltpu.sync_copy(mask_hbm.at[pl.ds(nbase * _K, _NPW * _K)], mask_v)
    pltpu.sync_copy(tsi_hbm.at[pl.ds(nbase, _NPW)], tsi_v)
    pltpu.sync_copy(w_hbm, w_v)

    pltpu.async_copy(ts_hbm.at[tsi_v], tsv_v, sem_ts).wait()
    pltpu.sync_copy(tsv_v, out_ts.at[pl.ds(nbase, _NPW)])

    rows_bufs = (rows0, rows1)
    obufs = (ob0, ob1)
    sems = (sem0, sem1)

    def _fire(b, t):
        pltpu.make_async_copy(
            msg_hbm.at[ridx_v.at[pl.ds(b * _ROWS, _ROWS)]],
            rows_bufs[t], sems[t]).start()

    def _wait(t):
        pltpu.make_async_copy(
            msg_hbm.at[ridx_v.at[pl.ds(0, _ROWS)]],
            rows_bufs[t], sems[t]).wait()

    w = [w_v[i] for i in range(32)]

    def _compute(b, t):
        rows = rows_bufs[t]
        ob = obufs[t]
        mv = [mask_v[pl.ds(b * _ROWS + 16 * i, 16)] for i in range(3)]
        for j in range(_NB):
            m = [jnp.full((16,), mv[(j * _K + s) // 16][(j * _K + s) % 16],
                          jnp.float32)
                 for s in range(_K)]

            def body(c, carry):
                sl = pl.ds(c * 16, 16)
                x = [rows[j * _K + s, sl] * m[s] for s in range(_K)]
                h = []
                for jj in range(3):
                    acc = w[18 + jj]
                    for s in range(_K):
                        acc = acc + x[s] * w[s * 3 + jj]
                    h.append(jnp.maximum(acc, 0.0))
                g = []
                for kk in range(2):
                    acc = w[27 + kk]
                    for jj in range(3):
                        acc = acc + h[jj] * w[21 + jj * 2 + kk]
                    g.append(jnp.maximum(acc, 0.0))
                o = w[31] + g[0] * w[29] + g[1] * w[30]
                ob[j, sl] = o
                return carry

            lax.fori_loop(0, _C16, body, 0)
        pltpu.sync_copy(ob, out_msg.at[pl.ds(nbase + b * _NB, _NB)])

    _fire(0, 0)

    def outer(i, carry):
        b0 = i * 2
        _wait(0)
        _fire(b0 + 1, 1)
        _compute(b0, 0)
        b1 = b0 + 1
        _wait(1)

        @pl.when(b1 + 1 < _NBATCH)
        def _():
            _fire(b1 + 1, 0)

        _compute(b1, 1)
        return carry

    lax.fori_loop(0, _NBATCH // 2, outer, 0)


_BSTRIDE = 3120
_BWIN = 3136
_TAILLO = _N - _BWIN


def _lane_gather(v, idx):
    dn = lax.GatherDimensionNumbers(offset_dims=(), collapsed_slice_dims=(0,),
                                    start_index_map=(0,))
    return lax.gather(v, idx[:, None], dn, (1,),
                      mode=lax.GatherScatterMode.PROMISE_IN_BOUNDS)


@functools.partial(
    pl.kernel,
    out_type=jax.ShapeDtypeStruct((_NW, _NODES_PAD), jnp.int32),
    mesh=_mesh,
    compiler_params=pltpu.CompilerParams(use_tc_tiling_on_sc=False,
                                        needs_layout_passes=False),
    scratch_types=[
        pltpu.VMEM((_BWIN,), jnp.int32),
        pltpu.VMEM((_NODES_PAD,), jnp.int32),
    ],
)
def _sc_boundaries(seg_hbm, out_hbm, seg_v, loc_v):
    wid = lax.axis_index("s") * 2 + lax.axis_index("c")
    iota = lax.iota(jnp.int32, 16)

    def init(i, c):
        loc_v[pl.ds(i * 16, 16)] = jnp.full((16,), -1, jnp.int32)
        return c

    lax.fori_loop(0, _NODES_PAD // 16, init, 0)

    def scan_window(lo):
        pltpu.sync_copy(seg_hbm.at[pl.ds(lo, _BWIN)], seg_v)

        def body(k, prev):
            cur = seg_v[pl.ds(k * 16, 16)]
            rotl = _lane_gather(prev, (iota + 1) & 15)
            nxt = jnp.where(iota < 15, rotl, _lane_gather(cur, iota * 0))
            m = prev != nxt
            ivec = iota + (lo + (k - 1) * 16)
            plsc.store_scatter(loc_v, [prev], ivec, mask=m)
            return cur

        return lax.fori_loop(1, _BWIN // 16, body, seg_v[pl.ds(0, 16)])

    scan_window(wid * _BSTRIDE)

    @pl.when(wid == _NW - 1)
    def _():
        last_vreg = scan_window(_TAILLO)
        rotl = _lane_gather(last_vreg, (iota + 1) & 15)
        m = (last_vreg != rotl) & (iota < 15)
        ivec = iota + (_N - 16)
        plsc.store_scatter(loc_v, [last_vreg], ivec, mask=m)
        plsc.store_scatter(loc_v, [_lane_gather(last_vreg, iota * 0 + 15)],
                           iota * 0 + (_N - 1), mask=iota < 1)

    pltpu.sync_copy(loc_v, out_hbm.at[wid])


_TB = 1024


def _tc_transpose_body(x_ref, o_ref):
    o_ref[...] = jnp.transpose(x_ref[...], (1, 0))


def _tc_row_majorize(m_t):
    grid = (pl.cdiv(_N, _TB),)
    return pl.pallas_call(
        _tc_transpose_body,
        grid=grid,
        in_specs=[pl.BlockSpec((_D, _TB), lambda i: (0, i))],
        out_specs=pl.BlockSpec((_TB, _D), lambda i: (i, 0)),
        out_shape=jax.ShapeDtypeStruct((_N, _D), jnp.float32),
    )(m_t)


_OB = 512


def _tc_col_majorize(out_rm):
    grid = (pl.cdiv(_NUM_NODES, _OB),)
    return pl.pallas_call(
        _tc_transpose_body,
        grid=grid,
        in_specs=[pl.BlockSpec((_OB, _D), lambda i: (i, 0))],
        out_specs=pl.BlockSpec((_D, _OB), lambda i: (0, i)),
        out_shape=jax.ShapeDtypeStruct((_D, _NUM_NODES), jnp.float32),
    )(out_rm)


def kernel(messages, timestamps, segment_ids, W1, b1, W2, b2, W3, b3):
    lastb = jnp.max(_sc_boundaries(segment_ids), axis=0)
    last_sem = lax.cummax(lastb, axis=0)
    ts_idx = jnp.clip(last_sem, 0, _N - 1)
    first = jnp.concatenate([jnp.full((1,), -1, jnp.int32),
                             last_sem[:-1]]) + 1
    cnt = jnp.where(lastb >= 0, jnp.minimum(lastb - first + 1, _K), 0)
    s = jnp.arange(_K, dtype=jnp.int32)
    rows = jnp.clip(lastb[:, None] - (_K - 1 - s)[None, :], 0, _N - 1)
    validf = ((_K - 1 - s)[None, :] < cnt[:, None]).astype(jnp.float32)

    row_idx = rows.reshape(-1)
    maskf = validf.reshape(-1)
    tsi = ts_idx

    wflat = jnp.concatenate([W1.reshape(-1), b1, W2.reshape(-1), b2,
                             W3.reshape(-1), b3])
    wrep = jnp.tile(wflat[:, None], (1, 16))

    msg_tbl = _tc_row_majorize(messages.T)
    out_msg, out_ts = _sc_aggregate(msg_tbl, timestamps, row_idx, tsi,
                                    maskf, wrep)
    return _tc_col_majorize(out_msg).T, out_ts[:_NUM_NODES]


# speedup vs baseline: 3.0405x; 1.0152x over previous
import functools

import jax
import jax.numpy as jnp
from jax import lax
from jax.experimental import pallas as pl
from jax.experimental.pallas import tpu as pltpu
from jax.experimental.pallas import tpu_sc as plsc

_N = 100000
_NUM_NODES = 10000
_D = 688
_K = 6
_NW = 32
_NODES_PAD = 10240
_NPW = _NODES_PAD // _NW
_NB = 8
_NBATCH = _NPW // _NB
_ROWS = _NB * _K
_C16 = _D // 16

_mesh = plsc.VectorSubcoreMesh(core_axis_name="c", subcore_axis_name="s")


@functools.partial(
    pl.kernel,
    out_type=(
        jax.ShapeDtypeStruct((_NODES_PAD, _D), jnp.float32),
        jax.ShapeDtypeStruct((_NODES_PAD,), jnp.float32),
    ),
    mesh=_mesh,
    compiler_params=pltpu.CompilerParams(use_tc_tiling_on_sc=False,
                                        needs_layout_passes=False),
    scratch_types=[
        pltpu.VMEM((_NPW * _K,), jnp.int32),
        pltpu.VMEM((_NPW * _K,), jnp.float32),
        pltpu.VMEM((_NPW,), jnp.int32),
        pltpu.VMEM((_NPW,), jnp.float32),
        pltpu.VMEM((32, 16), jnp.float32),
        pltpu.VMEM((_ROWS, _D), jnp.float32),
        pltpu.VMEM((_ROWS, _D), jnp.float32),
        pltpu.VMEM((_NB, _D), jnp.float32),
        pltpu.VMEM((_NB, _D), jnp.float32),
        pltpu.SemaphoreType.DMA,
        pltpu.SemaphoreType.DMA,
        pltpu.SemaphoreType.DMA,
    ],
)
def _sc_aggregate(msg_hbm, ts_hbm, ridx_hbm, tsi_hbm, mask_hbm, w_hbm,
                  out_msg, out_ts,
                  ridx_v, mask_v, tsi_v, tsv_v, w_v,
                  rows0, rows1, ob0, ob1, sem0, sem1, sem_ts):
    wid = lax.axis_index("s") * 2 + lax.axis_index("c")
    nbase = wid * _NPW

    pltpu.sync_copy(ridx_hbm.at[pl.ds(nbase * _K, _NPW * _K)], ridx_v)
    pltpu.sync_copy(mask_hbm.at[pl.ds(nbase * _K, _NPW * _K)], mask_v)
    pltpu.sync_copy(tsi_hbm.at[pl.ds(nbase, _NPW)], tsi_v)
    pltpu.sync_copy(w_hbm, w_v)

    pltpu.async_copy(ts_hbm.at[tsi_v], tsv_v, sem_ts).wait()
    pltpu.sync_copy(tsv_v, out_ts.at[pl.ds(nbase, _NPW)])

    rows_bufs = (rows0, rows1)
    obufs = (ob0, ob1)
    sems = (sem0, sem1)

    def _fire(b, t):
        pltpu.make_async_copy(
            msg_hbm.at[ridx_v.at[pl.ds(b * _ROWS, _ROWS)]],
            rows_bufs[t], sems[t]).start()

    def _wait(t):
        pltpu.make_async_copy(
            msg_hbm.at[ridx_v.at[pl.ds(0, _ROWS)]],
            rows_bufs[t], sems[t]).wait()

    w = [w_v[i] for i in range(32)]

    def _compute(b, t):
        rows = rows_bufs[t]
        ob = obufs[t]
        mv = [mask_v[pl.ds(b * _ROWS + 16 * i, 16)] for i in range(3)]
        for j in range(_NB):
            m = [jnp.full((16,), mv[(j * _K + s) // 16][(j * _K + s) % 16],
                          jnp.float32)
                 for s in range(_K)]

            def body(c, carry):
                sl = pl.ds(c * 16, 16)
                x = [rows[j * _K + s, sl] * m[s] for s in range(_K)]
                h = []
                for jj in range(3):
                    acc = w[18 + jj]
                    for s in range(_K):
                        acc = acc + x[s] * w[s * 3 + jj]
                    h.append(jnp.maximum(acc, 0.0))
                g = []
                for kk in range(2):
                    acc = w[27 + kk]
                    for jj in range(3):
                        acc = acc + h[jj] * w[21 + jj * 2 + kk]
                    g.append(jnp.maximum(acc, 0.0))
                o = w[31] + g[0] * w[29] + g[1] * w[30]
                ob[j, sl] = o
                return carry

            lax.fori_loop(0, _C16, body, 0)
        pltpu.sync_copy(ob, out_msg.at[pl.ds(nbase + b * _NB, _NB)])

    _fire(0, 0)

    def outer(i, carry):
        b0 = i * 2
        _wait(0)
        _fire(b0 + 1, 1)
        _compute(b0, 0)
        b1 = b0 + 1
        _wait(1)

        @pl.when(b1 + 1 < _NBATCH)
        def _():
            _fire(b1 + 1, 0)

        _compute(b1, 1)
        return carry

    lax.fori_loop(0, _NBATCH // 2, outer, 0)


_BSTRIDE = 3120
_BWIN = 3136
_TAILLO = _N - _BWIN


def _lane_gather(v, idx):
    dn = lax.GatherDimensionNumbers(offset_dims=(), collapsed_slice_dims=(0,),
                                    start_index_map=(0,))
    return lax.gather(v, idx[:, None], dn, (1,),
                      mode=lax.GatherScatterMode.PROMISE_IN_BOUNDS)


@functools.partial(
    pl.kernel,
    out_type=jax.ShapeDtypeStruct((_NW, _NODES_PAD), jnp.int32),
    mesh=_mesh,
    compiler_params=pltpu.CompilerParams(use_tc_tiling_on_sc=False,
                                        needs_layout_passes=False),
    scratch_types=[
        pltpu.VMEM((_BWIN,), jnp.int32),
        pltpu.VMEM((_NODES_PAD,), jnp.int32),
    ],
)
def _sc_boundaries(seg_hbm, out_hbm, seg_v, loc_v):
    wid = lax.axis_index("s") * 2 + lax.axis_index("c")
    iota = lax.iota(jnp.int32, 16)

    def init(i, c):
        loc_v[pl.ds(i * 16, 16)] = jnp.full((16,), -1, jnp.int32)
        return c

    lax.fori_loop(0, _NODES_PAD // 16, init, 0)

    def scan_window(lo):
        pltpu.sync_copy(seg_hbm.at[pl.ds(lo, _BWIN)], seg_v)

        def body(k, prev):
            cur = seg_v[pl.ds(k * 16, 16)]
            rotl = _lane_gather(prev, (iota + 1) & 15)
            nxt = jnp.where(iota < 15, rotl, _lane_gather(cur, iota * 0))
            m = prev != nxt
            ivec = iota + (lo + (k - 1) * 16)
            plsc.store_scatter(loc_v, [prev], ivec, mask=m)
            return cur

        return lax.fori_loop(1, _BWIN // 16, body, seg_v[pl.ds(0, 16)])

    scan_window(wid * _BSTRIDE)

    @pl.when(wid == _NW - 1)
    def _():
        last_vreg = scan_window(_TAILLO)
        rotl = _lane_gather(last_vreg, (iota + 1) & 15)
        m = (last_vreg != rotl) & (iota < 15)
        ivec = iota + (_N - 16)
        plsc.store_scatter(loc_v, [last_vreg], ivec, mask=m)
        plsc.store_scatter(loc_v, [_lane_gather(last_vreg, iota * 0 + 15)],
                           iota * 0 + (_N - 1), mask=iota < 1)

    pltpu.sync_copy(loc_v, out_hbm.at[wid])


_TB = 2048


def _tc_transpose_body(x_ref, o_ref):
    o_ref[...] = jnp.transpose(x_ref[...], (1, 0))


def _tc_row_majorize(m_t):
    grid = (pl.cdiv(_N, _TB),)
    return pl.pallas_call(
        _tc_transpose_body,
        grid=grid,
        in_specs=[pl.BlockSpec((_D, _TB), lambda i: (0, i))],
        out_specs=pl.BlockSpec((_TB, _D), lambda i: (i, 0)),
        out_shape=jax.ShapeDtypeStruct((_N, _D), jnp.float32),
    )(m_t)


_OB = 512


def _tc_col_majorize(out_rm):
    grid = (pl.cdiv(_NUM_NODES, _OB),)
    return pl.pallas_call(
        _tc_transpose_body,
        grid=grid,
        in_specs=[pl.BlockSpec((_OB, _D), lambda i: (i, 0))],
        out_specs=pl.BlockSpec((_D, _OB), lambda i: (0, i)),
        out_shape=jax.ShapeDtypeStruct((_D, _NUM_NODES), jnp.float32),
    )(out_rm)


def kernel(messages, timestamps, segment_ids, W1, b1, W2, b2, W3, b3):
    lastb = jnp.max(_sc_boundaries(segment_ids), axis=0)
    last_sem = lax.cummax(lastb, axis=0)
    ts_idx = jnp.clip(last_sem, 0, _N - 1)
    first = jnp.concatenate([jnp.full((1,), -1, jnp.int32),
                             last_sem[:-1]]) + 1
    cnt = jnp.where(lastb >= 0, jnp.minimum(lastb - first + 1, _K), 0)
    s = jnp.arange(_K, dtype=jnp.int32)
    rows = jnp.clip(lastb[:, None] - (_K - 1 - s)[None, :], 0, _N - 1)
    validf = ((_K - 1 - s)[None, :] < cnt[:, None]).astype(jnp.float32)

    row_idx = rows.reshape(-1)
    maskf = validf.reshape(-1)
    tsi = ts_idx

    wflat = jnp.concatenate([W1.reshape(-1), b1, W2.reshape(-1), b2,
                             W3.reshape(-1), b3])
    wrep = jnp.tile(wflat[:, None], (1, 16))

    msg_tbl = _tc_row_majorize(messages.T)
    out_msg, out_ts = _sc_aggregate(msg_tbl, timestamps, row_idx, tsi,
                                    maskf, wrep)
    return _tc_col_majorize(out_msg).T, out_ts[:_NUM_NODES]
